# baseline (speedup 1.0000x reference)
import functools

import jax
import jax.numpy as jnp
from jax import lax
from jax.experimental import pallas as pl
from jax.experimental.pallas import tpu as pltpu
from jax.experimental.pallas import tpu_sc as plsc

VOCAB = 1000000
EMBED_DIM = 64
AUX_DIM = 128
TARGET_DIM = 64
BATCH = 16384
HIST = 50

HALF = BATCH // 2
IDX_PER_DMA = 128
DMAS_PER_L = 4


def _make_sc_gather():
    info = plsc.get_sparse_core_info()
    nc, ns = info.num_cores, info.num_subcores
    nw = nc * ns
    b_per_w = BATCH // nw
    mesh = plsc.VectorSubcoreMesh(core_axis_name="c", subcore_axis_name="s")

    @functools.partial(
        pl.kernel,
        mesh=mesh,
        out_type=jax.ShapeDtypeStruct((HIST, HALF, 2 * EMBED_DIM), jnp.float32),
        scratch_types=[
            pltpu.VMEM((HIST, b_per_w), jnp.int32),
            pltpu.VMEM((b_per_w, 2 * EMBED_DIM), jnp.float32),
            pltpu.SemaphoreType.DMA,
        ],
        compiler_params=pltpu.CompilerParams(use_tc_tiling_on_sc=False),
    )
    def gather_k(tablep_hbm, idxt_hbm, g_hbm, idx_v, rows_v, sem):
        wid = lax.axis_index("s") * nc + lax.axis_index("c")
        half = wid // (nw // 2)
        rs = (wid % (nw // 2)) * b_per_w
        b0 = half * HALF + rs
        cs = half * EMBED_DIM
        pltpu.sync_copy(idxt_hbm.at[:, pl.ds(b0, b_per_w)], idx_v)

        def body(l, carry):
            handles = []
            for j in range(DMAS_PER_L):
                h = pltpu.async_copy(
                    tablep_hbm.at[idx_v.at[l, pl.ds(j * IDX_PER_DMA, IDX_PER_DMA)]],
                    rows_v.at[pl.ds(j * IDX_PER_DMA, IDX_PER_DMA)],
                    sem,
                )
                handles.append(h)
            for h in handles:
                h.wait()
            pltpu.sync_copy(
                rows_v.at[pl.ds(0, b_per_w), pl.ds(0, EMBED_DIM)],
                g_hbm.at[l, pl.ds(rs, b_per_w), pl.ds(cs, EMBED_DIM)],
            )
            return carry

        lax.fori_loop(0, HIST, body, 0)

    return gather_k


_sc_gather = _make_sc_gather()


def _mm_body(x_ref, w1_ref, w2_ref, b1_ref, b2_ref, o_ref):
    wct = jnp.dot(w2_ref[...], w1_ref[...], preferred_element_type=jnp.float32)
    bct = jnp.dot(w2_ref[...], b1_ref[...], preferred_element_type=jnp.float32)
    bct = bct + b2_ref[...]
    z = jnp.zeros((TARGET_DIM, EMBED_DIM), jnp.float32)
    bdct = jnp.concatenate(
        [jnp.concatenate([wct, z], axis=1), jnp.concatenate([z, wct], axis=1)],
        axis=0,
    )
    bbct = jnp.concatenate([bct, bct], axis=0)
    x = x_ref[0]
    o2t = lax.dot_general(
        bdct, x, dimension_numbers=(((1,), (1,)), ((), ())),
        preferred_element_type=jnp.float32,
    )
    o2t = o2t + bbct
    o_ref[0, :, 0:HALF] = o2t[0:TARGET_DIM, :]
    o_ref[0, :, HALF:BATCH] = o2t[TARGET_DIM:2 * TARGET_DIM, :]


def _tc_project(g, w1, w2, b1c, b2c):
    return pl.pallas_call(
        _mm_body,
        grid=(HIST,),
        in_specs=[
            pl.BlockSpec((1, HALF, 2 * EMBED_DIM), lambda l: (l, 0, 0)),
            pl.BlockSpec((AUX_DIM, EMBED_DIM), lambda l: (0, 0)),
            pl.BlockSpec((TARGET_DIM, AUX_DIM), lambda l: (0, 0)),
            pl.BlockSpec((AUX_DIM, 1), lambda l: (0, 0)),
            pl.BlockSpec((TARGET_DIM, 1), lambda l: (0, 0)),
        ],
        out_specs=pl.BlockSpec((1, TARGET_DIM, BATCH), lambda l: (l, 0, 0)),
        out_shape=jax.ShapeDtypeStruct((HIST, TARGET_DIM, BATCH), jnp.float32),
    )(g, w1, w2, b1c, b2c)


def kernel(indices, table, W1, b1, W2, b2):
    idx_t = indices.astype(jnp.int32).T
    table_p = jnp.pad(table, ((0, 0), (0, EMBED_DIM)))
    g = _sc_gather(table_p, idx_t)
    out_t = _tc_project(
        g, W1, W2, b1.reshape(AUX_DIM, 1), b2.reshape(TARGET_DIM, 1)
    )
    return jnp.transpose(out_t, (2, 0, 1))


# speedup vs baseline: 1.3526x; 1.3526x over previous
import functools

import jax
import jax.numpy as jnp
from jax import lax
from jax.experimental import pallas as pl
from jax.experimental.pallas import tpu as pltpu
from jax.experimental.pallas import tpu_sc as plsc

VOCAB = 1000000
EMBED_DIM = 64
AUX_DIM = 128
TARGET_DIM = 64
BATCH = 16384
HIST = 50

HALF = BATCH // 2
IDX_PER_DMA = 128
DMAS_PER_L = 4
VBLK = 8192


def _transform_body(xt_ref, w1_ref, w2_ref, b1_ref, b2_ref, o_ref):
    wc = lax.dot_general(
        w1_ref[...], w2_ref[...], dimension_numbers=(((0,), (1,)), ((), ())),
        preferred_element_type=jnp.float32,
    )
    bc = lax.dot_general(
        b1_ref[...], w2_ref[...], dimension_numbers=(((1,), (1,)), ((), ())),
        preferred_element_type=jnp.float32,
    ) + b2_ref[...]
    rows = lax.dot_general(
        xt_ref[...], wc, dimension_numbers=(((0,), (0,)), ((), ())),
        preferred_element_type=jnp.float32,
    )
    o_ref[:, 0:EMBED_DIM] = rows + bc
    o_ref[:, EMBED_DIM:2 * EMBED_DIM] = jnp.zeros((VBLK, EMBED_DIM), jnp.float32)


def _tc_transform(table_t, w1, w2, b1r, b2r):
    grid = (pl.cdiv(VOCAB, VBLK),)
    return pl.pallas_call(
        _transform_body,
        grid=grid,
        in_specs=[
            pl.BlockSpec((EMBED_DIM, VBLK), lambda i: (0, i)),
            pl.BlockSpec((AUX_DIM, EMBED_DIM), lambda i: (0, 0)),
            pl.BlockSpec((TARGET_DIM, AUX_DIM), lambda i: (0, 0)),
            pl.BlockSpec((1, AUX_DIM), lambda i: (0, 0)),
            pl.BlockSpec((1, TARGET_DIM), lambda i: (0, 0)),
        ],
        out_specs=pl.BlockSpec((VBLK, 2 * EMBED_DIM), lambda i: (i, 0)),
        out_shape=jax.ShapeDtypeStruct((VOCAB, 2 * EMBED_DIM), jnp.float32),
    )(table_t, w1, w2, b1r, b2r)


def _make_sc_gather():
    info = plsc.get_sparse_core_info()
    nc, ns = info.num_cores, info.num_subcores
    nw = nc * ns
    b_per_w = BATCH // nw
    mesh = plsc.VectorSubcoreMesh(core_axis_name="c", subcore_axis_name="s")

    @functools.partial(
        pl.kernel,
        mesh=mesh,
        out_type=jax.ShapeDtypeStruct((HIST, HALF, 2 * EMBED_DIM), jnp.float32),
        scratch_types=[
            pltpu.VMEM((HIST, b_per_w), jnp.int32),
            pltpu.VMEM((b_per_w, 2 * EMBED_DIM), jnp.float32),
            pltpu.SemaphoreType.DMA,
        ],
        compiler_params=pltpu.CompilerParams(use_tc_tiling_on_sc=False),
    )
    def gather_k(tablep_hbm, idxt_hbm, g_hbm, idx_v, rows_v, sem):
        wid = lax.axis_index("s") * nc + lax.axis_index("c")
        half = wid // (nw // 2)
        rs = (wid % (nw // 2)) * b_per_w
        b0 = half * HALF + rs
        cs = half * EMBED_DIM
        pltpu.sync_copy(idxt_hbm.at[:, pl.ds(b0, b_per_w)], idx_v)

        def body(l, carry):
            handles = []
            for j in range(DMAS_PER_L):
                h = pltpu.async_copy(
                    tablep_hbm.at[idx_v.at[l, pl.ds(j * IDX_PER_DMA, IDX_PER_DMA)]],
                    rows_v.at[pl.ds(j * IDX_PER_DMA, IDX_PER_DMA)],
                    sem,
                )
                handles.append(h)
            for h in handles:
                h.wait()
            pltpu.sync_copy(
                rows_v.at[pl.ds(0, b_per_w), pl.ds(0, EMBED_DIM)],
                g_hbm.at[l, pl.ds(rs, b_per_w), pl.ds(cs, EMBED_DIM)],
            )
            return carry

        lax.fori_loop(0, HIST, body, 0)

    return gather_k


_sc_gather = _make_sc_gather()


def _transpose_body(x_ref, o_ref):
    x = x_ref[0]
    r = lax.broadcasted_iota(jnp.int32, (AUX_DIM, AUX_DIM), 0)
    c = lax.broadcasted_iota(jnp.int32, (AUX_DIM, AUX_DIM), 1)
    eye = jnp.where(r == c, 1.0, 0.0).astype(jnp.float32)
    o2t = lax.dot_general(
        eye, x, dimension_numbers=(((1,), (1,)), ((), ())),
        preferred_element_type=jnp.float32,
    )
    o_ref[0, :, 0:HALF] = o2t[0:TARGET_DIM, :]
    o_ref[0, :, HALF:BATCH] = o2t[TARGET_DIM:2 * TARGET_DIM, :]


def _tc_transpose(g):
    return pl.pallas_call(
        _transpose_body,
        grid=(HIST,),
        in_specs=[pl.BlockSpec((1, HALF, 2 * EMBED_DIM), lambda l: (l, 0, 0))],
        out_specs=pl.BlockSpec((1, TARGET_DIM, BATCH), lambda l: (l, 0, 0)),
        out_shape=jax.ShapeDtypeStruct((HIST, TARGET_DIM, BATCH), jnp.float32),
    )(g)


def kernel(indices, table, W1, b1, W2, b2):
    idx_t = indices.astype(jnp.int32).T
    t2p = _tc_transform(
        table.T,
        W1, W2, b1.reshape(1, AUX_DIM), b2.reshape(1, TARGET_DIM),
    )
    g = _sc_gather(t2p, idx_t)
    out_t = _tc_transpose(g)
    return jnp.transpose(out_t, (2, 0, 1))
